# Initial kernel scaffold; baseline (speedup 1.0000x reference)
import jax
import jax.numpy as jnp
from jax.experimental import pallas as pl


def kernel(x_game, x_state, ei_gg, et_gg, ei_hist, ea_hist, ei_in, ei_pp, params):
    raise NotImplementedError("write your pallas kernel here")


# speedup vs baseline: 1.3097x; 1.3097x over previous
import functools

import jax
import jax.numpy as jnp
from jax import lax
from jax.experimental import pallas as pl
from jax.experimental.pallas import tpu as pltpu
from jax.experimental.pallas import tpu_sc as plsc

NGN = 50000
NSN = 50000
EE = 800000
NCORE = 2
NSUBC = 16
NWORK = NCORE * NSUBC
CH = 128
NCHUNK = EE // CH
CHUNK_BASE = NCHUNK // NWORK
CHUNK_EXTRA = NCHUNK - CHUNK_BASE * NWORK
BSZ = 1568
NOUT = NWORK * BSZ
ADUMP = 1600
CAP = 1024
LBUF = NWORK * CAP
TOTCAP = NWORK * LBUF

_f32 = jnp.float32
_i32 = jnp.int32


def _mesh():
    return plsc.VectorSubcoreMesh(core_axis_name="c", subcore_axis_name="s",
                                  num_cores=NCORE, num_subcores=NSUBC)


def _wid():
    return lax.axis_index("c") * NSUBC + lax.axis_index("s")


def _nchunks(wid):
    return jnp.where(wid < CHUNK_EXTRA, CHUNK_BASE + 1, CHUNK_BASE)


def _zero_i32(buf, n16):
    def zr(i, carry):
        buf[pl.ds(i * 16, 16)] = jnp.zeros((16,), _i32)
        return carry
    lax.fori_loop(0, n16, zr, 0)


def _zero_f32(buf, n16):
    def zr(i, carry):
        buf[pl.ds(i * 16, 16)] = jnp.zeros((16,), _f32)
        return carry
    lax.fori_loop(0, n16, zr, 0)


def _bucket(d):
    return ((d >> 5) * 21400) >> 20


@functools.lru_cache(maxsize=None)
def _sc_partition():
    @functools.partial(
        pl.kernel,
        out_type=(pltpu.HBM((TOTCAP,), _i32),
                  pltpu.HBM((TOTCAP,), _i32),
                  pltpu.HBM((TOTCAP,), _i32)),
        mesh=_mesh(),
        compiler_params=pltpu.CompilerParams(needs_layout_passes=False,
                                             use_tc_tiling_on_sc=False),
        scratch_types=[
            pltpu.VMEM((CH,), _i32), pltpu.VMEM((CH,), _i32),
            pltpu.VMEM((CH,), _i32),
            pltpu.VMEM((LBUF,), _i32), pltpu.VMEM((LBUF,), _i32),
            pltpu.VMEM((LBUF,), _i32),
            pltpu.VMEM((512,), _i32),
            pltpu.VMEM((512,), _i32),
        ],
    )
    def k(src, dst, ext, srcP, dstlP, extP,
          sb, db, eb, lsrc, ldst, lext, hist, cur):
        wid = _wid()
        lane = lax.iota(_i32, 16)

        _zero_i32(hist, 32)

        def hist_pass(j, carry):
            off = (wid + j * NWORK) * CH
            pltpu.sync_copy(dst.at[pl.ds(off, CH)], db)

            def grp(g, carry2):
                d = db[pl.ds(g * 16, 16)]
                b = _bucket(d)
                plsc.addupdate_scatter(hist, [b * 16 + lane],
                                       jnp.ones((16,), _i32))
                return carry2
            lax.fori_loop(0, CH // 16, grp, 0)
            return carry
        lax.fori_loop(0, _nchunks(wid), hist_pass, 0)

        for b in range(NWORK):
            h16 = hist[pl.ds(b * 16, 16)]
            excl = plsc.cumsum(h16) - h16
            cur[pl.ds(b * 16, 16)] = excl + b * CAP

        _zero_i32(lsrc, LBUF // 16)
        _zero_i32(lext, LBUF // 16)

        def fill_dump(i, carry):
            ldst[pl.ds(i * 16, 16)] = jnp.full((16,), BSZ, _i32)
            return carry
        lax.fori_loop(0, LBUF // 16, fill_dump, 0)

        def scat_pass(j, carry):
            off = (wid + j * NWORK) * CH
            pltpu.sync_copy(src.at[pl.ds(off, CH)], sb)
            pltpu.sync_copy(dst.at[pl.ds(off, CH)], db)
            pltpu.sync_copy(ext.at[pl.ds(off, CH)], eb)

            def grp(g, carry2):
                sl = pl.ds(g * 16, 16)
                d = db[sl]
                b = _bucket(d)
                dl = d - b * BSZ
                key = b * 16 + lane
                pos = plsc.load_gather(cur, [key])
                pos = jnp.minimum(pos, (b + 1) * CAP - 1)
                plsc.store_scatter(lsrc, [pos], sb[sl])
                plsc.store_scatter(ldst, [pos], dl)
                plsc.store_scatter(lext, [pos], eb[sl])
                plsc.store_scatter(cur, [key], pos + 1)
                return carry2
            lax.fori_loop(0, CH // 16, grp, 0)
            return carry
        lax.fori_loop(0, _nchunks(wid), scat_pass, 0)

        for b in range(NWORK):
            gb = (b * NWORK + wid) * CAP
            pltpu.sync_copy(lsrc.at[pl.ds(b * CAP, CAP)],
                            srcP.at[pl.ds(gb, CAP)])
            pltpu.sync_copy(ldst.at[pl.ds(b * CAP, CAP)],
                            dstlP.at[pl.ds(gb, CAP)])
            pltpu.sync_copy(lext.at[pl.ds(b * CAP, CAP)],
                            extP.at[pl.ds(gb, CAP)])

    return k


@functools.lru_cache(maxsize=None)
def _sc_segsum(width, nrel):
    nacc = nrel * ADUMP

    @functools.partial(
        pl.kernel,
        out_type=pltpu.HBM((nrel, NOUT * width), _f32),
        mesh=_mesh(),
        compiler_params=pltpu.CompilerParams(needs_layout_passes=False,
                                             use_tc_tiling_on_sc=False),
        scratch_types=[
            pltpu.VMEM((CH,), _i32), pltpu.VMEM((CH,), _i32),
            pltpu.VMEM((CH,), _i32),
            pltpu.VMEM((CH, width), _f32),
            pltpu.VMEM((nacc * width,), _f32),
            pltpu.SemaphoreType.DMA,
        ],
    )
    def k(table, srcP, dstlP, extP, out,
          sidx, didx, eidx, rows, acc, sem):
        b = _wid()
        lane16 = lax.iota(_i32, 16)

        _zero_f32(acc, nacc * width // 16)
        if True:
            base = b * NWORK * CAP
            nch = NWORK * CAP // CH

            def chunk(j, carry, base=base):
                off = base + j * CH
                pltpu.sync_copy(srcP.at[pl.ds(off, CH)], sidx)
                pltpu.sync_copy(dstlP.at[pl.ds(off, CH)], didx)
                if nrel > 1:
                    pltpu.sync_copy(extP.at[pl.ds(off, CH)], eidx)
                pltpu.async_copy(table.at[sidx], rows, sem).wait()

                def grp16(g, carry2):
                    dlv = didx[pl.ds(g * 16, 16)]
                    if nrel > 1:
                        etv = eidx[pl.ds(g * 16, 16)]
                        rbase = (etv * ADUMP + dlv) * width
                    else:
                        rbase = dlv * width
                    rbvs = [rbase + w * 16 for w in range(width // 16)]
                    for l in range(16):
                        e = g * 16 + l
                        for w in range(width // 16):
                            plsc.addupdate_scatter(
                                acc, [lane16 + rbvs[w][l]],
                                rows[e, pl.ds(w * 16, 16)])
                    return carry2
                lax.fori_loop(0, CH // 16, grp16, 0)
                return carry
            lax.fori_loop(0, nch, chunk, 0)

        for r in range(nrel):
            for h in range(2):
                pltpu.sync_copy(
                    acc.at[pl.ds((r * ADUMP + h * 784) * width, 784 * width)],
                    out.at[r, pl.ds((b * BSZ + h * 784) * width,
                                    784 * width)])

    return k


@functools.lru_cache(maxsize=None)
def _sc_count(has_et):
    width = 16

    @functools.partial(
        pl.kernel,
        out_type=pltpu.HBM((NOUT * width,), _f32),
        mesh=_mesh(),
        compiler_params=pltpu.CompilerParams(needs_layout_passes=False,
                                             use_tc_tiling_on_sc=False),
        scratch_types=[
            pltpu.VMEM((CH,), _i32), pltpu.VMEM((CH,), _i32),
            pltpu.VMEM((ADUMP * width,), _f32),
        ],
    )
    def k(dstlP, extP, out, didx, eidx, acc):
        b = _wid()
        lane = lax.iota(_i32, 16)
        ones = jnp.ones((16,), _f32)
        msk = lane < (2 if has_et else 1)

        _zero_f32(acc, ADUMP * width // 16)
        if True:
            base = b * NWORK * CAP
            nch = NWORK * CAP // CH

            def chunk(j, carry, base=base):
                off = base + j * CH
                pltpu.sync_copy(dstlP.at[pl.ds(off, CH)], didx)
                if has_et:
                    pltpu.sync_copy(extP.at[pl.ds(off, CH)], eidx)

                def grp16(g, carry2):
                    dlv = didx[pl.ds(g * 16, 16)]
                    if has_et:
                        etv = eidx[pl.ds(g * 16, 16)]
                    for l in range(16):
                        rb = dlv[l] * width
                        if has_et:
                            ch = jnp.where(lane == 0, 0, 1 + etv[l]) + rb
                        else:
                            ch = jnp.zeros((16,), _i32) + rb
                        plsc.addupdate_scatter(acc, [ch], ones, mask=msk)
                    return carry2
                lax.fori_loop(0, CH // 16, grp16, 0)
                return carry
            lax.fori_loop(0, nch, chunk, 0)

        for h in range(2):
            pltpu.sync_copy(acc.at[pl.ds(h * 784 * width, 784 * width)],
                            out.at[pl.ds((b * BSZ + h * 784) * width,
                                         784 * width)])

    return k


@functools.lru_cache(maxsize=None)
def _sc_resgated():
    width = 32

    @functools.partial(
        pl.kernel,
        out_type=pltpu.HBM((NOUT * width,), _f32),
        mesh=_mesh(),
        compiler_params=pltpu.CompilerParams(needs_layout_passes=False,
                                             use_tc_tiling_on_sc=False),
        scratch_types=[
            pltpu.VMEM((CH,), _i32), pltpu.VMEM((CH,), _i32),
            pltpu.VMEM((CH,), _i32), pltpu.VMEM((CH,), _i32),
            pltpu.VMEM((CH, width), _f32), pltpu.VMEM((CH, width), _f32),
            pltpu.VMEM((CH, width), _f32), pltpu.VMEM((CH, width), _f32),
            pltpu.VMEM((ADUMP * width,), _f32),
            pltpu.SemaphoreType.DMA,
        ],
    )
    def k(kt, qt, vt, et, srcP, dstlP, extP, out,
          sidx, didx, gidx, eidx, krows, qrows, vrows, erows,
          acc, sem):
        b = _wid()
        lane16 = lax.iota(_i32, 16)

        _zero_f32(acc, ADUMP * width // 16)
        if True:
            base = b * NWORK * CAP
            nch = NWORK * CAP // CH

            def chunk(j, carry, base=base):
                off = base + j * CH
                pltpu.sync_copy(srcP.at[pl.ds(off, CH)], sidx)
                pltpu.sync_copy(dstlP.at[pl.ds(off, CH)], didx)
                pltpu.sync_copy(extP.at[pl.ds(off, CH)], eidx)

                def grp(g, carry2):
                    sl = pl.ds(g * 16, 16)
                    dl = didx[sl]
                    gidx[sl] = b * BSZ + jnp.minimum(dl, BSZ - 1)
                    return carry2
                lax.fori_loop(0, CH // 16, grp, 0)

                d1 = pltpu.async_copy(kt.at[gidx], krows, sem)
                d2 = pltpu.async_copy(qt.at[sidx], qrows, sem)
                d3 = pltpu.async_copy(vt.at[sidx], vrows, sem)
                d4 = pltpu.async_copy(et.at[eidx], erows, sem)
                d1.wait(); d2.wait(); d3.wait(); d4.wait()

                def grp16(g, carry2):
                    dlv = didx[pl.ds(g * 16, 16)]
                    rbvs = [dlv * width + w * 16 for w in range(width // 16)]
                    for l in range(16):
                        e = g * 16 + l
                        for w in range(width // 16):
                            sl = pl.ds(w * 16, 16)
                            z = krows[e, sl] + qrows[e, sl] + erows[e, sl]
                            gate = 1.0 / (1.0 + jnp.exp(-z))
                            plsc.addupdate_scatter(acc,
                                                   [lane16 + rbvs[w][l]],
                                                   gate * vrows[e, sl])
                    return carry2
                lax.fori_loop(0, CH // 16, grp16, 0)
                return carry
            lax.fori_loop(0, nch, chunk, 0)

        for h in range(2):
            pltpu.sync_copy(acc.at[pl.ds(h * 784 * width, 784 * width)],
                            out.at[pl.ds((b * BSZ + h * 784) * width,
                                         784 * width)])

    return k


def _rowspec(shape, br):
    if len(shape) == 2:
        return pl.BlockSpec((br, shape[1]), lambda i: (i, 0))
    return pl.BlockSpec((shape[0], br, shape[2]), lambda i: (0, i, 0))


def _fullspec(shape):
    nd = len(shape)
    return pl.BlockSpec(shape, lambda i, nd=nd: (0,) * nd)


def _tc(body, row_ins, full_ins, out_sd, br=2000, n=NSN):
    grid = (n // br,)
    in_specs = ([_rowspec(a.shape, br) for a in row_ins]
                + [_fullspec(a.shape) for a in full_ins])
    out_shape = [jax.ShapeDtypeStruct(s, d) for s, d in out_sd]
    out_specs = [_rowspec(s, br) for s, d in out_sd]
    return pl.pallas_call(body, grid=grid, in_specs=in_specs,
                          out_specs=out_specs, out_shape=out_shape)(
        *row_ins, *full_ins)


def _cat(h):
    return jnp.concatenate([h[0], h[1]], axis=1)


def _halves(x):
    return jnp.stack([x[:, :32], x[:, 32:]])


def _l2(o):
    nrm = jnp.sqrt(jnp.sum(o * o, axis=1, keepdims=True))
    return o / jnp.maximum(nrm, 1e-12)


def kernel(x_game, x_state, ei_gg, et_gg, ei_hist, ea_hist, ei_in, ei_pp, params):
    p = params

    def b2(name):
        return p[name].reshape(1, -1)

    part = _sc_partition()
    orig = jnp.arange(EE, dtype=_i32)
    sp_gg = part(ei_gg[0], ei_gg[1], et_gg)
    sp_h = part(ei_hist[0], ei_hist[1], orig)
    sp_i = part(ei_in[0], ei_in[1], orig)
    sp_p = part(ei_pp[0], ei_pp[1], orig)

    def runc(has_et, sp, n):
        return _sc_count(has_et)(sp[1], sp[2]).reshape(NOUT, 16)[:n]

    cnt_gg = runc(True, sp_gg, NGN)
    cnt_h = runc(False, sp_h, NSN)
    cnt_i = runc(False, sp_i, NSN)
    cnt_p = runc(False, sp_p, NSN)

    def norm_body(cg, chh, cii, cpp,
                  dinvg_o, invr_o, invch_o, invci_o, dinvp_o, invcp_o):
        deg = cg[...][:, :1]
        dinvg_o[...] = jnp.where(deg > 0,
                                 1.0 / jnp.sqrt(jnp.maximum(deg, 1e-12)), 0.0)
        invr_o[...] = 1.0 / jnp.maximum(cg[...][:, 1:4], 1.0)
        invch_o[...] = 1.0 / jnp.maximum(chh[...][:, :1], 1.0)
        invci_o[...] = 1.0 / jnp.maximum(cii[...][:, :1], 1.0)
        degp = cpp[...][:, :1]
        dinvp_o[...] = jnp.where(degp > 0,
                                 1.0 / jnp.sqrt(jnp.maximum(degp, 1e-12)), 0.0)
        invcp_o[...] = 1.0 / jnp.maximum(degp, 1.0)

    dinv_gg, invr, invch, invci, dinv_pp, invcp = _tc(
        norm_body, [cnt_gg, cnt_h, cnt_i, cnt_p], [],
        [((NGN, 1), _f32), ((NGN, 3), _f32), ((NSN, 1), _f32),
         ((NSN, 1), _f32), ((NSN, 1), _f32), ((NSN, 1), _f32)])

    seg16_1 = _sc_segsum(16, 1)
    seg16_3 = _sc_segsum(16, 3)
    seg32 = _sc_segsum(32, 1)

    def run16(table, sp):
        r = seg16_1(table, sp[0], sp[1], sp[2])
        return r.reshape(NOUT, 16)[:NGN]

    def run32(table, sp, n):
        r = seg32(table, sp[0], sp[1], sp[2])
        return r.reshape(NOUT, 32)[:n]

    def prep1_body(x_ref, dv_ref, o_ref):
        u = x_ref[...] * dv_ref[...]
        o_ref[...] = jnp.concatenate(
            [u, jnp.zeros((u.shape[0], 9), _f32)], axis=1)

    (u0,) = _tc(prep1_body, [x_game, dinv_gg], [], [((NGN, 16), _f32)])
    S1 = run16(u0, sp_gg)

    def hop16_body(s_ref, dv_ref, h_o, u_o):
        hh = s_ref[...] * dv_ref[...]
        h_o[...] = hh
        u_o[...] = hh * dv_ref[...]

    h1, u1 = _tc(hop16_body, [S1, dinv_gg], [],
                 [((NGN, 16), _f32), ((NGN, 16), _f32)])
    S2 = run16(u1, sp_gg)
    h2, _ = _tc(hop16_body, [S2, dinv_gg], [],
                [((NGN, 16), _f32), ((NGN, 16), _f32)])

    W1p = jnp.zeros((16, 64), _f32).at[:7].set(p['tag1_W'][1])
    W2p = jnp.zeros((16, 64), _f32).at[:7].set(p['tag1_W'][2])

    def t1o_body(x_ref, h1_ref, h2_ref, W0, W1, W2, b, g_o, q_o):
        out = (x_ref[...] @ W0[...] + h1_ref[...] @ W1[...]
               + h2_ref[...] @ W2[...] + b[...])
        g_o[...] = _halves(out)
        q_o[...] = jnp.stack([out[:, 0:16], out[:, 16:32],
                              out[:, 32:48], out[:, 48:64]])

    g_h, g_q = _tc(t1o_body, [x_game, h1, h2],
                   [p['tag1_W'][0], W1p, W2p, b2('tag1_b')],
                   [((2, NGN, 32), _f32), ((4, NGN, 16), _f32)])

    Srg = [seg16_3(g_q[q], sp_gg[0], sp_gg[1],
                   sp_gg[2]).reshape(3, NOUT, 16)[:, :NGN]
           for q in range(4)]

    def rgcn_body(g_ref, s0, s1, s2, s3, invr_ref, Wroot, Wr, b, o_ref):
        g = _cat(g_ref)
        acc = g @ Wroot[...] + b[...]
        iv = invr_ref[...]
        for r in range(3):
            mean = jnp.concatenate(
                [s0[r], s1[r], s2[r], s3[r]], axis=1) * iv[:, r:r + 1]
            acc = acc + mean @ Wr[r]
        o_ref[...] = _halves(acc)

    (g2_h,) = _tc(rgcn_body,
                  [g_h, Srg[0], Srg[1], Srg[2], Srg[3], invr],
                  [p['rgcn_Wroot'], p['rgcn_Wr'], b2('rgcn_b')],
                  [((2, NGN, 32), _f32)])

    def mm7_body(x_ref, W, b, o_ref):
        o_ref[...] = _halves(x_ref[...] @ W[...] + b[...])

    def mm64_body(x_ref, W, b, o_ref):
        o_ref[...] = _halves(_cat(x_ref) @ W[...] + b[...])

    (k_h,) = _tc(mm7_body, [x_state], [p['rg_Wk'], b2('rg_bk')],
                 [((2, NSN, 32), _f32)])
    (q_h,) = _tc(mm64_body, [g2_h], [p['rg_Wq'], b2('rg_bq')],
                 [((2, NGN, 32), _f32)])
    (v_h,) = _tc(mm64_body, [g2_h], [p['rg_Wv'], b2('rg_bv')],
                 [((2, NGN, 32), _f32)])
    (e_h,) = _tc(mm7_body, [ea_hist], [p['rg_We'], b2('rg_be')],
                 [((2, EE, 32), _f32)], br=8000, n=EE)

    resg = _sc_resgated()
    kpad = [jnp.zeros((NOUT, 32), _f32).at[:NSN].set(k_h[hh])
            for hh in range(2)]
    Ares = [resg(kpad[hh], q_h[hh], v_h[hh], e_h[hh],
                 sp_h[0], sp_h[1],
                 sp_h[2]).reshape(NOUT, 32)[:NSN] for hh in range(2)]
    As3 = [run32(g2_h[hh], sp_h, NSN) for hh in range(2)]

    def res_s3_body(ar0, ar1, as0, as1, x_ref, invch_ref,
                    Wskip, rgb, W3l, b3l, W3r, o_ref):
        agg = jnp.concatenate([ar0[...], ar1[...]], axis=1)
        s_res = agg + x_ref[...] @ Wskip[...] + rgb[...]
        mean = jnp.concatenate([as0[...], as1[...]], axis=1) * invch_ref[...]
        o = mean @ W3l[...] + b3l[...] + s_res @ W3r[...]
        o_ref[...] = _halves(_l2(o))

    (s_h,) = _tc(res_s3_body,
                 [Ares[0], Ares[1], As3[0], As3[1], x_state, invch],
                 [p['rg_Wskip'], b2('rg_b'), p['sage3_Wl'], b2('sage3_bl'),
                  p['sage3_Wr']],
                 [((2, NSN, 32), _f32)])

    A4 = [run32(g2_h[hh], sp_i, NSN) for hh in range(2)]

    def s45_body(a0, a1, s_ref, invci_ref, dinvp_ref,
                 W4l, b4l, W4r, W5l, b5l, W5r, s5_o, u_o):
        mean = jnp.concatenate([a0[...], a1[...]], axis=1) * invci_ref[...]
        s4 = _l2(mean @ W4l[...] + b4l[...] + _cat(s_ref) @ W4r[...])
        s5 = _l2(mean @ W5l[...] + b5l[...] + s4 @ W5r[...])
        s5_o[...] = _halves(s5)
        u_o[...] = _halves(s5 * dinvp_ref[...])

    s5_h, ut_h = _tc(s45_body, [A4[0], A4[1], s_h, invci, dinv_pp],
                     [p['sage4_Wl'], b2('sage4_bl'), p['sage4_Wr'],
                      p['sage5_Wl'], b2('sage5_bl'), p['sage5_Wr']],
                     [((2, NSN, 32), _f32), ((2, NSN, 32), _f32)])

    def hop32_body(a0, a1, dv_ref, h_o, u_o):
        d = dv_ref[...]
        hs, us = [], []
        for pr in (a0, a1):
            hh = pr[...] * d
            hs.append(hh)
            us.append(hh * d)
        h_o[...] = jnp.stack(hs)
        u_o[...] = jnp.stack(us)

    hu = ut_h
    hs_list = []
    for _hop in range(3):
        Aa = [run32(hu[hh], sp_p, NSN) for hh in range(2)]
        hh_, hu = _tc(hop32_body, [Aa[0], Aa[1], dinv_pp], [],
                      [((2, NSN, 32), _f32), ((2, NSN, 32), _f32)])
        hs_list.append(hh_)

    def t2o_body(s_ref, h1_ref, h2_ref, h3_ref, W, b, o_ref):
        out = (_cat(s_ref) @ W[0] + _cat(h1_ref) @ W[1]
               + _cat(h2_ref) @ W[2] + _cat(h3_ref) @ W[3] + b[...])
        o_ref[...] = _halves(out)

    (t_h,) = _tc(t2o_body, [s5_h, hs_list[0], hs_list[1], hs_list[2]],
                 [p['tag2_W'], b2('tag2_b')], [((2, NSN, 32), _f32)])

    A6 = [run32(t_h[hh], sp_p, NSN) for hh in range(2)]

    def s6_body(a0, a1, t_ref, invcp_ref, W6l, b6l, W6r, o_ref):
        mean = jnp.concatenate([a0[...], a1[...]], axis=1) * invcp_ref[...]
        o = mean @ W6l[...] + b6l[...] + _cat(t_ref) @ W6r[...]
        o_ref[...] = _l2(o)

    (s6,) = _tc(s6_body, [A6[0], A6[1], t_h, invcp],
                [p['sage6_Wl'], b2('sage6_bl'), p['sage6_Wr']],
                [((NSN, 64), _f32)])

    def head_body(x_ref, lw, lb, fw, fb, o_ref):
        h = jnp.maximum(x_ref[...] @ lw[...] + lb[...], 0.0)
        o_ref[...] = h @ fw[...] + fb[...]

    (logits,) = _tc(head_body, [s6],
                    [p['lin_W'], b2('lin_b'), p['last_W'], b2('last_b')],
                    [((NSN, 1), _f32)])

    def ls_body(x_ref, o_ref):
        x = x_ref[...]
        m = jnp.max(x)
        o_ref[...] = x - m - jnp.log(jnp.sum(jnp.exp(x - m)))

    lpad = jnp.concatenate(
        [logits[:, 0], jnp.full((392 * 128 - NSN,), -1e30, _f32)])
    ls = pl.pallas_call(
        ls_body, out_shape=jax.ShapeDtypeStruct((392, 128), _f32))(
        lpad.reshape(392, 128))
    return ls.reshape(392 * 128)[:NSN].reshape(NSN, 1)


# speedup vs baseline: 1.3143x; 1.0035x over previous
import functools

import jax
import jax.numpy as jnp
from jax import lax
from jax.experimental import pallas as pl
from jax.experimental.pallas import tpu as pltpu
from jax.experimental.pallas import tpu_sc as plsc

NGN = 50000
NSN = 50000
EE = 800000
NCORE = 2
NSUBC = 16
NWORK = NCORE * NSUBC
CH = 128
NCHUNK = EE // CH
CHUNK_BASE = NCHUNK // NWORK
CHUNK_EXTRA = NCHUNK - CHUNK_BASE * NWORK
BSZ = 1568
NOUT = NWORK * BSZ
ADUMP = 1600
CAP = 1024
LBUF = NWORK * CAP
TOTCAP = NWORK * LBUF

_f32 = jnp.float32
_i32 = jnp.int32


def _mesh():
    return plsc.VectorSubcoreMesh(core_axis_name="c", subcore_axis_name="s",
                                  num_cores=NCORE, num_subcores=NSUBC)


def _wid():
    return lax.axis_index("c") * NSUBC + lax.axis_index("s")


def _nchunks(wid):
    return jnp.where(wid < CHUNK_EXTRA, CHUNK_BASE + 1, CHUNK_BASE)


def _zero_i32(buf, n16):
    def zr(i, carry):
        buf[pl.ds(i * 16, 16)] = jnp.zeros((16,), _i32)
        return carry
    lax.fori_loop(0, n16, zr, 0)


def _zero_f32(buf, n16):
    def zr(i, carry):
        buf[pl.ds(i * 16, 16)] = jnp.zeros((16,), _f32)
        return carry
    lax.fori_loop(0, n16, zr, 0)


def _bucket(d):
    return ((d >> 5) * 21400) >> 20


@functools.lru_cache(maxsize=None)
def _sc_partition():
    @functools.partial(
        pl.kernel,
        out_type=(pltpu.HBM((TOTCAP,), _i32),
                  pltpu.HBM((TOTCAP,), _i32),
                  pltpu.HBM((TOTCAP,), _i32)),
        mesh=_mesh(),
        compiler_params=pltpu.CompilerParams(needs_layout_passes=False,
                                             use_tc_tiling_on_sc=False),
        scratch_types=[
            pltpu.VMEM((CH,), _i32), pltpu.VMEM((CH,), _i32),
            pltpu.VMEM((CH,), _i32),
            pltpu.VMEM((LBUF,), _i32), pltpu.VMEM((LBUF,), _i32),
            pltpu.VMEM((LBUF,), _i32),
            pltpu.VMEM((512,), _i32),
            pltpu.VMEM((512,), _i32),
        ],
    )
    def k(src, dst, ext, srcP, dstlP, extP,
          sb, db, eb, lsrc, ldst, lext, hist, cur):
        wid = _wid()
        lane = lax.iota(_i32, 16)

        _zero_i32(hist, 32)

        def hist_pass(j, carry):
            off = (wid + j * NWORK) * CH
            pltpu.sync_copy(dst.at[pl.ds(off, CH)], db)

            def grp(g, carry2):
                d = db[pl.ds(g * 16, 16)]
                b = _bucket(d)
                plsc.addupdate_scatter(hist, [b * 16 + lane],
                                       jnp.ones((16,), _i32))
                return carry2
            lax.fori_loop(0, CH // 16, grp, 0)
            return carry
        lax.fori_loop(0, _nchunks(wid), hist_pass, 0)

        for b in range(NWORK):
            h16 = hist[pl.ds(b * 16, 16)]
            excl = plsc.cumsum(h16) - h16
            cur[pl.ds(b * 16, 16)] = excl + b * CAP

        _zero_i32(lsrc, LBUF // 16)
        _zero_i32(lext, LBUF // 16)

        def fill_dump(i, carry):
            ldst[pl.ds(i * 16, 16)] = jnp.full((16,), BSZ, _i32)
            return carry
        lax.fori_loop(0, LBUF // 16, fill_dump, 0)

        def scat_pass(j, carry):
            off = (wid + j * NWORK) * CH
            pltpu.sync_copy(src.at[pl.ds(off, CH)], sb)
            pltpu.sync_copy(dst.at[pl.ds(off, CH)], db)
            pltpu.sync_copy(ext.at[pl.ds(off, CH)], eb)

            def grp(g, carry2):
                sl = pl.ds(g * 16, 16)
                d = db[sl]
                b = _bucket(d)
                dl = d - b * BSZ
                key = b * 16 + lane
                pos = plsc.load_gather(cur, [key])
                pos = jnp.minimum(pos, (b + 1) * CAP - 1)
                plsc.store_scatter(lsrc, [pos], sb[sl])
                plsc.store_scatter(ldst, [pos], dl)
                plsc.store_scatter(lext, [pos], eb[sl])
                plsc.store_scatter(cur, [key], pos + 1)
                return carry2
            lax.fori_loop(0, CH // 16, grp, 0)
            return carry
        lax.fori_loop(0, _nchunks(wid), scat_pass, 0)

        for b in range(NWORK):
            gb = (b * NWORK + wid) * CAP
            pltpu.sync_copy(lsrc.at[pl.ds(b * CAP, CAP)],
                            srcP.at[pl.ds(gb, CAP)])
            pltpu.sync_copy(ldst.at[pl.ds(b * CAP, CAP)],
                            dstlP.at[pl.ds(gb, CAP)])
            pltpu.sync_copy(lext.at[pl.ds(b * CAP, CAP)],
                            extP.at[pl.ds(gb, CAP)])

    return k


@functools.lru_cache(maxsize=None)
def _sc_segsum(width, nrel):
    nacc = nrel * ADUMP

    @functools.partial(
        pl.kernel,
        out_type=pltpu.HBM((nrel, NOUT * width), _f32),
        mesh=_mesh(),
        compiler_params=pltpu.CompilerParams(needs_layout_passes=False,
                                             use_tc_tiling_on_sc=False),
        scratch_types=[
            pltpu.VMEM((2, CH), _i32), pltpu.VMEM((2, CH), _i32),
            pltpu.VMEM((2, CH), _i32),
            pltpu.VMEM((2, CH, width), _f32),
            pltpu.VMEM((nacc * width,), _f32),
            pltpu.SemaphoreType.DMA,
        ],
    )
    def k(table, srcP, dstlP, extP, out,
          sidx, didx, eidx, rows, acc, sem):
        b = _wid()
        lane16 = lax.iota(_i32, 16)

        _zero_f32(acc, nacc * width // 16)
        if True:
            base = b * NWORK * CAP
            nch = NWORK * CAP // CH
            maxoff = TOTCAP - CH

            def load_idx(j, p):
                off = jnp.minimum(base + j * CH, maxoff)
                pltpu.sync_copy(srcP.at[pl.ds(off, CH)], sidx.at[p])
                pltpu.sync_copy(dstlP.at[pl.ds(off, CH)], didx.at[p])
                if nrel > 1:
                    pltpu.sync_copy(extP.at[pl.ds(off, CH)], eidx.at[p])

            load_idx(0, 0)
            pltpu.async_copy(table.at[sidx.at[0]], rows.at[0], sem)

            def chunk2(jj, carry):
                for p in range(2):
                    j = jj * 2 + p
                    q = 1 - p
                    load_idx(j + 1, q)
                    pltpu.make_async_copy(
                        table.at[sidx.at[p]], rows.at[p], sem).wait()
                    pltpu.async_copy(table.at[sidx.at[q]], rows.at[q], sem)

                    def grp16(g, carry2, p=p):
                        dlv = didx[p, pl.ds(g * 16, 16)]
                        if nrel > 1:
                            etv = eidx[p, pl.ds(g * 16, 16)]
                            rbase = (etv * ADUMP + dlv) * width
                        else:
                            rbase = dlv * width
                        rbvs = [rbase + w * 16 for w in range(width // 16)]
                        for l in range(16):
                            e = g * 16 + l
                            for w in range(width // 16):
                                plsc.addupdate_scatter(
                                    acc, [lane16 + rbvs[w][l]],
                                    rows[p, e, pl.ds(w * 16, 16)])
                        return carry2
                    lax.fori_loop(0, CH // 16, grp16, 0)
                return carry
            lax.fori_loop(0, nch // 2, chunk2, 0)
            pltpu.make_async_copy(
                table.at[sidx.at[0]], rows.at[0], sem).wait()

        for r in range(nrel):
            for h in range(2):
                pltpu.sync_copy(
                    acc.at[pl.ds((r * ADUMP + h * 784) * width, 784 * width)],
                    out.at[r, pl.ds((b * BSZ + h * 784) * width,
                                    784 * width)])

    return k


@functools.lru_cache(maxsize=None)
def _sc_count(has_et):
    width = 16

    @functools.partial(
        pl.kernel,
        out_type=pltpu.HBM((NOUT * width,), _f32),
        mesh=_mesh(),
        compiler_params=pltpu.CompilerParams(needs_layout_passes=False,
                                             use_tc_tiling_on_sc=False),
        scratch_types=[
            pltpu.VMEM((CH,), _i32), pltpu.VMEM((CH,), _i32),
            pltpu.VMEM((ADUMP * width,), _f32),
        ],
    )
    def k(dstlP, extP, out, didx, eidx, acc):
        b = _wid()
        lane = lax.iota(_i32, 16)
        ones = jnp.ones((16,), _f32)
        msk = lane < (2 if has_et else 1)

        _zero_f32(acc, ADUMP * width // 16)
        if True:
            base = b * NWORK * CAP
            nch = NWORK * CAP // CH

            def chunk(j, carry, base=base):
                off = base + j * CH
                pltpu.sync_copy(dstlP.at[pl.ds(off, CH)], didx)
                if has_et:
                    pltpu.sync_copy(extP.at[pl.ds(off, CH)], eidx)

                def grp16(g, carry2):
                    dlv = didx[pl.ds(g * 16, 16)]
                    if has_et:
                        etv = eidx[pl.ds(g * 16, 16)]
                    for l in range(16):
                        rb = dlv[l] * width
                        if has_et:
                            ch = jnp.where(lane == 0, 0, 1 + etv[l]) + rb
                        else:
                            ch = jnp.zeros((16,), _i32) + rb
                        plsc.addupdate_scatter(acc, [ch], ones, mask=msk)
                    return carry2
                lax.fori_loop(0, CH // 16, grp16, 0)
                return carry
            lax.fori_loop(0, nch, chunk, 0)

        for h in range(2):
            pltpu.sync_copy(acc.at[pl.ds(h * 784 * width, 784 * width)],
                            out.at[pl.ds((b * BSZ + h * 784) * width,
                                         784 * width)])

    return k


@functools.lru_cache(maxsize=None)
def _sc_resgated():
    width = 32

    @functools.partial(
        pl.kernel,
        out_type=pltpu.HBM((NOUT * width,), _f32),
        mesh=_mesh(),
        compiler_params=pltpu.CompilerParams(needs_layout_passes=False,
                                             use_tc_tiling_on_sc=False),
        scratch_types=[
            pltpu.VMEM((CH,), _i32), pltpu.VMEM((CH,), _i32),
            pltpu.VMEM((CH,), _i32), pltpu.VMEM((CH,), _i32),
            pltpu.VMEM((CH, width), _f32), pltpu.VMEM((CH, width), _f32),
            pltpu.VMEM((CH, width), _f32), pltpu.VMEM((CH, width), _f32),
            pltpu.VMEM((ADUMP * width,), _f32),
            pltpu.SemaphoreType.DMA,
        ],
    )
    def k(kt, qt, vt, et, srcP, dstlP, extP, out,
          sidx, didx, gidx, eidx, krows, qrows, vrows, erows,
          acc, sem):
        b = _wid()
        lane16 = lax.iota(_i32, 16)

        _zero_f32(acc, ADUMP * width // 16)
        if True:
            base = b * NWORK * CAP
            nch = NWORK * CAP // CH

            def chunk(j, carry, base=base):
                off = base + j * CH
                pltpu.sync_copy(srcP.at[pl.ds(off, CH)], sidx)
                pltpu.sync_copy(dstlP.at[pl.ds(off, CH)], didx)
                pltpu.sync_copy(extP.at[pl.ds(off, CH)], eidx)

                def grp(g, carry2):
                    sl = pl.ds(g * 16, 16)
                    dl = didx[sl]
                    gidx[sl] = b * BSZ + jnp.minimum(dl, BSZ - 1)
                    return carry2
                lax.fori_loop(0, CH // 16, grp, 0)

                d1 = pltpu.async_copy(kt.at[gidx], krows, sem)
                d2 = pltpu.async_copy(qt.at[sidx], qrows, sem)
                d3 = pltpu.async_copy(vt.at[sidx], vrows, sem)
                d4 = pltpu.async_copy(et.at[eidx], erows, sem)
                d1.wait(); d2.wait(); d3.wait(); d4.wait()

                def grp16(g, carry2):
                    dlv = didx[pl.ds(g * 16, 16)]
                    rbvs = [dlv * width + w * 16 for w in range(width // 16)]
                    for l in range(16):
                        e = g * 16 + l
                        for w in range(width // 16):
                            sl = pl.ds(w * 16, 16)
                            z = krows[e, sl] + qrows[e, sl] + erows[e, sl]
                            gate = 1.0 / (1.0 + jnp.exp(-z))
                            plsc.addupdate_scatter(acc,
                                                   [lane16 + rbvs[w][l]],
                                                   gate * vrows[e, sl])
                    return carry2
                lax.fori_loop(0, CH // 16, grp16, 0)
                return carry
            lax.fori_loop(0, nch, chunk, 0)

        for h in range(2):
            pltpu.sync_copy(acc.at[pl.ds(h * 784 * width, 784 * width)],
                            out.at[pl.ds((b * BSZ + h * 784) * width,
                                         784 * width)])

    return k


def _rowspec(shape, br):
    if len(shape) == 2:
        return pl.BlockSpec((br, shape[1]), lambda i: (i, 0))
    return pl.BlockSpec((shape[0], br, shape[2]), lambda i: (0, i, 0))


def _fullspec(shape):
    nd = len(shape)
    return pl.BlockSpec(shape, lambda i, nd=nd: (0,) * nd)


def _tc(body, row_ins, full_ins, out_sd, br=2000, n=NSN):
    grid = (n // br,)
    in_specs = ([_rowspec(a.shape, br) for a in row_ins]
                + [_fullspec(a.shape) for a in full_ins])
    out_shape = [jax.ShapeDtypeStruct(s, d) for s, d in out_sd]
    out_specs = [_rowspec(s, br) for s, d in out_sd]
    return pl.pallas_call(body, grid=grid, in_specs=in_specs,
                          out_specs=out_specs, out_shape=out_shape)(
        *row_ins, *full_ins)


def _cat(h):
    return jnp.concatenate([h[0], h[1]], axis=1)


def _halves(x):
    return jnp.stack([x[:, :32], x[:, 32:]])


def _l2(o):
    nrm = jnp.sqrt(jnp.sum(o * o, axis=1, keepdims=True))
    return o / jnp.maximum(nrm, 1e-12)


def kernel(x_game, x_state, ei_gg, et_gg, ei_hist, ea_hist, ei_in, ei_pp, params):
    p = params

    def b2(name):
        return p[name].reshape(1, -1)

    part = _sc_partition()
    orig = jnp.arange(EE, dtype=_i32)
    sp_gg = part(ei_gg[0], ei_gg[1], et_gg)
    sp_h = part(ei_hist[0], ei_hist[1], orig)
    sp_i = part(ei_in[0], ei_in[1], orig)
    sp_p = part(ei_pp[0], ei_pp[1], orig)

    def runc(has_et, sp, n):
        return _sc_count(has_et)(sp[1], sp[2]).reshape(NOUT, 16)[:n]

    cnt_gg = runc(True, sp_gg, NGN)
    cnt_h = runc(False, sp_h, NSN)
    cnt_i = runc(False, sp_i, NSN)
    cnt_p = runc(False, sp_p, NSN)

    def norm_body(cg, chh, cii, cpp,
                  dinvg_o, invr_o, invch_o, invci_o, dinvp_o, invcp_o):
        deg = cg[...][:, :1]
        dinvg_o[...] = jnp.where(deg > 0,
                                 1.0 / jnp.sqrt(jnp.maximum(deg, 1e-12)), 0.0)
        invr_o[...] = 1.0 / jnp.maximum(cg[...][:, 1:4], 1.0)
        invch_o[...] = 1.0 / jnp.maximum(chh[...][:, :1], 1.0)
        invci_o[...] = 1.0 / jnp.maximum(cii[...][:, :1], 1.0)
        degp = cpp[...][:, :1]
        dinvp_o[...] = jnp.where(degp > 0,
                                 1.0 / jnp.sqrt(jnp.maximum(degp, 1e-12)), 0.0)
        invcp_o[...] = 1.0 / jnp.maximum(degp, 1.0)

    dinv_gg, invr, invch, invci, dinv_pp, invcp = _tc(
        norm_body, [cnt_gg, cnt_h, cnt_i, cnt_p], [],
        [((NGN, 1), _f32), ((NGN, 3), _f32), ((NSN, 1), _f32),
         ((NSN, 1), _f32), ((NSN, 1), _f32), ((NSN, 1), _f32)])

    seg16_1 = _sc_segsum(16, 1)
    seg16_3 = _sc_segsum(16, 3)
    seg32 = _sc_segsum(32, 1)

    def run16(table, sp):
        r = seg16_1(table, sp[0], sp[1], sp[2])
        return r.reshape(NOUT, 16)[:NGN]

    def run32(table, sp, n):
        r = seg32(table, sp[0], sp[1], sp[2])
        return r.reshape(NOUT, 32)[:n]

    def prep1_body(x_ref, dv_ref, o_ref):
        u = x_ref[...] * dv_ref[...]
        o_ref[...] = jnp.concatenate(
            [u, jnp.zeros((u.shape[0], 9), _f32)], axis=1)

    (u0,) = _tc(prep1_body, [x_game, dinv_gg], [], [((NGN, 16), _f32)])
    S1 = run16(u0, sp_gg)

    def hop16_body(s_ref, dv_ref, h_o, u_o):
        hh = s_ref[...] * dv_ref[...]
        h_o[...] = hh
        u_o[...] = hh * dv_ref[...]

    h1, u1 = _tc(hop16_body, [S1, dinv_gg], [],
                 [((NGN, 16), _f32), ((NGN, 16), _f32)])
    S2 = run16(u1, sp_gg)
    h2, _ = _tc(hop16_body, [S2, dinv_gg], [],
                [((NGN, 16), _f32), ((NGN, 16), _f32)])

    W1p = jnp.zeros((16, 64), _f32).at[:7].set(p['tag1_W'][1])
    W2p = jnp.zeros((16, 64), _f32).at[:7].set(p['tag1_W'][2])

    def t1o_body(x_ref, h1_ref, h2_ref, W0, W1, W2, b, g_o, q_o):
        out = (x_ref[...] @ W0[...] + h1_ref[...] @ W1[...]
               + h2_ref[...] @ W2[...] + b[...])
        g_o[...] = _halves(out)
        q_o[...] = jnp.stack([out[:, 0:16], out[:, 16:32],
                              out[:, 32:48], out[:, 48:64]])

    g_h, g_q = _tc(t1o_body, [x_game, h1, h2],
                   [p['tag1_W'][0], W1p, W2p, b2('tag1_b')],
                   [((2, NGN, 32), _f32), ((4, NGN, 16), _f32)])

    Srg = [seg16_3(g_q[q], sp_gg[0], sp_gg[1],
                   sp_gg[2]).reshape(3, NOUT, 16)[:, :NGN]
           for q in range(4)]

    def rgcn_body(g_ref, s0, s1, s2, s3, invr_ref, Wroot, Wr, b, o_ref):
        g = _cat(g_ref)
        acc = g @ Wroot[...] + b[...]
        iv = invr_ref[...]
        for r in range(3):
            mean = jnp.concatenate(
                [s0[r], s1[r], s2[r], s3[r]], axis=1) * iv[:, r:r + 1]
            acc = acc + mean @ Wr[r]
        o_ref[...] = _halves(acc)

    (g2_h,) = _tc(rgcn_body,
                  [g_h, Srg[0], Srg[1], Srg[2], Srg[3], invr],
                  [p['rgcn_Wroot'], p['rgcn_Wr'], b2('rgcn_b')],
                  [((2, NGN, 32), _f32)])

    def mm7_body(x_ref, W, b, o_ref):
        o_ref[...] = _halves(x_ref[...] @ W[...] + b[...])

    def mm64_body(x_ref, W, b, o_ref):
        o_ref[...] = _halves(_cat(x_ref) @ W[...] + b[...])

    (k_h,) = _tc(mm7_body, [x_state], [p['rg_Wk'], b2('rg_bk')],
                 [((2, NSN, 32), _f32)])
    (q_h,) = _tc(mm64_body, [g2_h], [p['rg_Wq'], b2('rg_bq')],
                 [((2, NGN, 32), _f32)])
    (v_h,) = _tc(mm64_body, [g2_h], [p['rg_Wv'], b2('rg_bv')],
                 [((2, NGN, 32), _f32)])
    (e_h,) = _tc(mm7_body, [ea_hist], [p['rg_We'], b2('rg_be')],
                 [((2, EE, 32), _f32)], br=8000, n=EE)

    resg = _sc_resgated()
    kpad = [jnp.zeros((NOUT, 32), _f32).at[:NSN].set(k_h[hh])
            for hh in range(2)]
    Ares = [resg(kpad[hh], q_h[hh], v_h[hh], e_h[hh],
                 sp_h[0], sp_h[1],
                 sp_h[2]).reshape(NOUT, 32)[:NSN] for hh in range(2)]
    As3 = [run32(g2_h[hh], sp_h, NSN) for hh in range(2)]

    def res_s3_body(ar0, ar1, as0, as1, x_ref, invch_ref,
                    Wskip, rgb, W3l, b3l, W3r, o_ref):
        agg = jnp.concatenate([ar0[...], ar1[...]], axis=1)
        s_res = agg + x_ref[...] @ Wskip[...] + rgb[...]
        mean = jnp.concatenate([as0[...], as1[...]], axis=1) * invch_ref[...]
        o = mean @ W3l[...] + b3l[...] + s_res @ W3r[...]
        o_ref[...] = _halves(_l2(o))

    (s_h,) = _tc(res_s3_body,
                 [Ares[0], Ares[1], As3[0], As3[1], x_state, invch],
                 [p['rg_Wskip'], b2('rg_b'), p['sage3_Wl'], b2('sage3_bl'),
                  p['sage3_Wr']],
                 [((2, NSN, 32), _f32)])

    A4 = [run32(g2_h[hh], sp_i, NSN) for hh in range(2)]

    def s45_body(a0, a1, s_ref, invci_ref, dinvp_ref,
                 W4l, b4l, W4r, W5l, b5l, W5r, s5_o, u_o):
        mean = jnp.concatenate([a0[...], a1[...]], axis=1) * invci_ref[...]
        s4 = _l2(mean @ W4l[...] + b4l[...] + _cat(s_ref) @ W4r[...])
        s5 = _l2(mean @ W5l[...] + b5l[...] + s4 @ W5r[...])
        s5_o[...] = _halves(s5)
        u_o[...] = _halves(s5 * dinvp_ref[...])

    s5_h, ut_h = _tc(s45_body, [A4[0], A4[1], s_h, invci, dinv_pp],
                     [p['sage4_Wl'], b2('sage4_bl'), p['sage4_Wr'],
                      p['sage5_Wl'], b2('sage5_bl'), p['sage5_Wr']],
                     [((2, NSN, 32), _f32), ((2, NSN, 32), _f32)])

    def hop32_body(a0, a1, dv_ref, h_o, u_o):
        d = dv_ref[...]
        hs, us = [], []
        for pr in (a0, a1):
            hh = pr[...] * d
            hs.append(hh)
            us.append(hh * d)
        h_o[...] = jnp.stack(hs)
        u_o[...] = jnp.stack(us)

    hu = ut_h
    hs_list = []
    for _hop in range(3):
        Aa = [run32(hu[hh], sp_p, NSN) for hh in range(2)]
        hh_, hu = _tc(hop32_body, [Aa[0], Aa[1], dinv_pp], [],
                      [((2, NSN, 32), _f32), ((2, NSN, 32), _f32)])
        hs_list.append(hh_)

    def t2o_body(s_ref, h1_ref, h2_ref, h3_ref, W, b, o_ref):
        out = (_cat(s_ref) @ W[0] + _cat(h1_ref) @ W[1]
               + _cat(h2_ref) @ W[2] + _cat(h3_ref) @ W[3] + b[...])
        o_ref[...] = _halves(out)

    (t_h,) = _tc(t2o_body, [s5_h, hs_list[0], hs_list[1], hs_list[2]],
                 [p['tag2_W'], b2('tag2_b')], [((2, NSN, 32), _f32)])

    A6 = [run32(t_h[hh], sp_p, NSN) for hh in range(2)]

    def s6_body(a0, a1, t_ref, invcp_ref, W6l, b6l, W6r, o_ref):
        mean = jnp.concatenate([a0[...], a1[...]], axis=1) * invcp_ref[...]
        o = mean @ W6l[...] + b6l[...] + _cat(t_ref) @ W6r[...]
        o_ref[...] = _l2(o)

    (s6,) = _tc(s6_body, [A6[0], A6[1], t_h, invcp],
                [p['sage6_Wl'], b2('sage6_bl'), p['sage6_Wr']],
                [((NSN, 64), _f32)])

    def head_body(x_ref, lw, lb, fw, fb, o_ref):
        h = jnp.maximum(x_ref[...] @ lw[...] + lb[...], 0.0)
        o_ref[...] = h @ fw[...] + fb[...]

    (logits,) = _tc(head_body, [s6],
                    [p['lin_W'], b2('lin_b'), p['last_W'], b2('last_b')],
                    [((NSN, 1), _f32)])

    def ls_body(x_ref, o_ref):
        x = x_ref[...]
        m = jnp.max(x)
        o_ref[...] = x - m - jnp.log(jnp.sum(jnp.exp(x - m)))

    lpad = jnp.concatenate(
        [logits[:, 0], jnp.full((392 * 128 - NSN,), -1e30, _f32)])
    ls = pl.pallas_call(
        ls_body, out_shape=jax.ShapeDtypeStruct((392, 128), _f32))(
        lpad.reshape(392, 128))
    return ls.reshape(392 * 128)[:NSN].reshape(NSN, 1)
